# baseline (speedup 1.0000x reference)
import functools

import jax
import jax.numpy as jnp
from jax import lax
from jax.experimental import pallas as pl
from jax.experimental.pallas import tpu as pltpu
from jax.experimental.pallas import tpu_sc as plsc

LAT_BINS = 100000
LON_BINS = 100000
EMBED_DIM = 64
BATCH = 16384

_info = plsc.get_sparse_core_info()
_NC = _info.num_cores
_NS = _info.num_subcores
_NW = _NC * _NS

_W = 1536
_NFULL = 65
_TAIL_X0 = _NFULL * _W
_TAIL_W = 256
_SEG = 4096
_NSEG = BATCH // _SEG
_C = 64
_TAIL_ON = True
_SCAN_ON = True
_CHUNK_ON = True
_CSTORE_ON = True


def _extract_row(slab_v, x, buf_v, k, half):
    xv = jnp.full((16,), x, dtype=jnp.int32)
    for q in range(EMBED_DIM // 16):
        cv = lax.iota(jnp.int32, 16) + 16 * q
        vals = plsc.load_gather(slab_v, [cv, xv])
        buf_v[k, pl.ds(half * EMBED_DIM + 16 * q, 16)] = vals


def _make_body(second_pass):
    def _body(idx_hbm, tt_hbm, *rest):
        if second_pass:
            (prev_hbm, out_hbm, slab_v, idxs_v, hitx_v, hitj_v, buf_v,
             jl_v, sem, sem2) = rest
        else:
            (out_hbm, slab_v, idxs_v, hitx_v, hitj_v, buf_v,
             jl_v, sem, sem2) = rest
            prev_hbm = None
        wid = lax.axis_index("s") * _NC + lax.axis_index("c")

        def do_slab(x0, width, slab_w):
            x0 = pl.multiple_of(x0, 128)
            cp = pltpu.async_copy(
                tt_hbm.at[:, pl.ds(x0, slab_w)],
                slab_v.at[:, pl.ds(0, slab_w)], sem)
            cp.wait()
            x1 = x0 + width

            def do_seg(g, carry):
                pltpu.sync_copy(idx_hbm.at[pl.ds(g * _SEG, _SEG)], idxs_v)

                def scan_step(t, cur):
                    v = idxs_v[pl.ds(16 * t, 16)]
                    m = (v >= x0) & (v < x1)
                    cnt = plsc.all_reduce_population_count(m)[0]
                    if _CSTORE_ON:
                        plsc.store_compressed(hitx_v.at[pl.ds(cur, 16)],
                                              v - x0, mask=m)
                        plsc.store_compressed(
                            hitj_v.at[pl.ds(cur, 16)],
                            lax.iota(jnp.int32, 16) + (g * _SEG + 16 * t),
                            mask=m)
                    return cur + cnt

                n = lax.fori_loop(0, _SEG // 16, scan_step, 0)

                def do_chunk(c2, carry2):
                    base = c2 * _C
                    xs = []
                    for kk in range(_C // 16):
                        off = base + 16 * kk + lax.iota(jnp.int32, 16)
                        ie = jnp.where(off < n, off, base)
                        jl_v[pl.ds(16 * kk, 16)] = plsc.load_gather(
                            hitj_v, [ie])
                        xs.append(plsc.load_gather(hitx_v, [ie]))
                    if second_pass:
                        pltpu.async_copy(prev_hbm.at[jl_v], buf_v, sem2).wait()
                    for k in range(_C):
                        x = xs[k // 16][k % 16]
                        _extract_row(slab_v, x, buf_v, k,
                                     1 if second_pass else 0)
                    pltpu.async_copy(buf_v, out_hbm.at[jl_v], sem2).wait()
                    return carry2

                if _CHUNK_ON:
                    lax.fori_loop(0, (n + _C - 1) // _C, do_chunk, 0)
                return carry

            if _SCAN_ON:
                lax.fori_loop(0, _NSEG, do_seg, 0)

        def slab_loop(si, carry):
            s = wid + si * _NW
            do_slab(s * _W, _W, _W)
            return carry

        n_slabs = jnp.where(wid < (_NFULL % _NW), _NFULL // _NW + 1,
                            _NFULL // _NW)
        lax.fori_loop(0, n_slabs, slab_loop, 0)

        if _TAIL_ON:
            @pl.when(wid == _NW - 1)
            def _():
                do_slab(_TAIL_X0, LAT_BINS - _TAIL_X0, _TAIL_W)

    return _body


def _sc_call(body, n_in, extra_in_types):
    mesh = plsc.VectorSubcoreMesh(core_axis_name="c", subcore_axis_name="s")
    return functools.partial(
        pl.kernel,
        mesh=mesh,
        out_type=jax.ShapeDtypeStruct((BATCH, 2 * EMBED_DIM), jnp.float32),
        compiler_params=pltpu.CompilerParams(needs_layout_passes=False),
        scratch_types=[
            pltpu.VMEM((EMBED_DIM, _W), jnp.float32),
            pltpu.VMEM((_SEG,), jnp.int32),
            pltpu.VMEM((_SEG + 16,), jnp.int32),
            pltpu.VMEM((_SEG + 16,), jnp.int32),
            pltpu.VMEM((_C, 2 * EMBED_DIM), jnp.float32),
            pltpu.VMEM((_C,), jnp.int32),
            pltpu.SemaphoreType.DMA,
            pltpu.SemaphoreType.DMA,
        ],
    )(body)


def kernel(latitudes, longitudes, lat_table, lon_table):
    k1 = _sc_call(_make_body(False), 2, [])
    o1 = k1(latitudes, lat_table.T)
    k2 = _sc_call(_make_body(True), 3, [])
    return k2(longitudes, lon_table.T, o1)


# speedup vs baseline: 1.3000x; 1.3000x over previous
import functools

import jax
import jax.numpy as jnp
from jax import lax
from jax.experimental import pallas as pl
from jax.experimental.pallas import tpu as pltpu
from jax.experimental.pallas import tpu_sc as plsc

LAT_BINS = 100000
LON_BINS = 100000
EMBED_DIM = 64
BATCH = 16384

_info = plsc.get_sparse_core_info()
_NC = _info.num_cores
_NS = _info.num_subcores
_NW = _NC * _NS

_LOGW = 9
_W = 1 << _LOGW
_NSLAB_FULL = 195
_TAIL_S = 195
_TAIL_W = 256
_C = 64
_HCAP = BATCH + 16


def _extract_row(slab_v, x, buf_v, k, half):
    xv = jnp.full((16,), x, dtype=jnp.int32)
    for q in range(EMBED_DIM // 16):
        cv = lax.iota(jnp.int32, 16) + 16 * q
        vals = plsc.load_gather(slab_v, [cv, xv])
        buf_v[k, pl.ds(half * EMBED_DIM + 16 * q, 16)] = vals


def _make_body(second_pass):
    def _body(idx_hbm, tt_hbm, *rest):
        if second_pass:
            (prev_hbm, out_hbm, slab_v, idxs_v, hitv_v, hitj_v, cx_v, cj_v,
             buf_v, jl_v, sem, sem2) = rest
        else:
            (out_hbm, slab_v, idxs_v, hitv_v, hitj_v, cx_v, cj_v,
             buf_v, jl_v, sem, sem2) = rest
            prev_hbm = None
        wid = lax.axis_index("s") * _NC + lax.axis_index("c")

        pltpu.sync_copy(idx_hbm, idxs_v)

        def scan_step(t, cur):
            for u in range(4):
                e = (4 * t + u) * 16
                v = idxs_v[pl.ds(e, 16)]
                m = ((lax.shift_right_logical(v, _LOGW) & (_NW - 1))
                     == jnp.full((16,), 0, jnp.int32) + wid)
                cnt = plsc.all_reduce_population_count(m)[0]
                plsc.store_compressed(hitv_v.at[pl.ds(cur, 16)], v, mask=m)
                plsc.store_compressed(
                    hitj_v.at[pl.ds(cur, 16)],
                    lax.iota(jnp.int32, 16) + e, mask=m)
                cur = cur + cnt
            return cur

        n_mine = lax.fori_loop(0, BATCH // 64, scan_step, 0)
        n_it = (n_mine + 15) // 16

        def do_slab(s_id, slab_w):
            cp = pltpu.async_copy(
                tt_hbm.at[:, pl.ds(pl.multiple_of(s_id * _W, 128), slab_w)],
                slab_v.at[:, pl.ds(0, slab_w)], sem)
            cp.wait()

            def filt(t, cur2):
                v = hitv_v[pl.ds(16 * t, 16)]
                j = hitj_v[pl.ds(16 * t, 16)]
                m2 = (lax.shift_right_logical(v, _LOGW) == (
                    jnp.full((16,), 0, jnp.int32) + s_id))
                m2 = m2 & ((lax.iota(jnp.int32, 16) + 16 * t) < n_mine)
                cnt2 = plsc.all_reduce_population_count(m2)[0]
                plsc.store_compressed(cx_v.at[pl.ds(cur2, 16)],
                                      v & (_W - 1), mask=m2)
                plsc.store_compressed(cj_v.at[pl.ds(cur2, 16)], j, mask=m2)
                return cur2 + cnt2

            n2 = lax.fori_loop(0, n_it, filt, 0)

            def do_chunk(c2, carry2):
                base = c2 * _C
                xs = []
                for kk in range(_C // 16):
                    off = base + 16 * kk + lax.iota(jnp.int32, 16)
                    ie = jnp.where(off < n2, off, base)
                    jl_v[pl.ds(16 * kk, 16)] = plsc.load_gather(cj_v, [ie])
                    xs.append(plsc.load_gather(cx_v, [ie]))
                if second_pass:
                    pltpu.async_copy(prev_hbm.at[jl_v], buf_v, sem2).wait()
                for k in range(_C):
                    x = xs[k // 16][k % 16]
                    _extract_row(slab_v, x, buf_v, k,
                                 1 if second_pass else 0)
                pltpu.async_copy(buf_v, out_hbm.at[jl_v], sem2).wait()
                return carry2

            lax.fori_loop(0, (n2 + _C - 1) // _C, do_chunk, 0)

        def slab_loop(si, carry):
            do_slab(wid + si * _NW, _W)
            return carry

        lax.fori_loop(0, 6, slab_loop, 0)

        @pl.when(wid < 3)
        def _():
            do_slab(192 + wid, _W)

        @pl.when(wid == 3)
        def _():
            do_slab(_TAIL_S, _TAIL_W)

    return _body


def _sc_call(body):
    mesh = plsc.VectorSubcoreMesh(core_axis_name="c", subcore_axis_name="s")
    return functools.partial(
        pl.kernel,
        mesh=mesh,
        out_type=jax.ShapeDtypeStruct((BATCH, 2 * EMBED_DIM), jnp.float32),
        compiler_params=pltpu.CompilerParams(needs_layout_passes=False),
        scratch_types=[
            pltpu.VMEM((EMBED_DIM, _W), jnp.float32),
            pltpu.VMEM((BATCH,), jnp.int32),
            pltpu.VMEM((_HCAP,), jnp.int32),
            pltpu.VMEM((_HCAP,), jnp.int32),
            pltpu.VMEM((_HCAP,), jnp.int32),
            pltpu.VMEM((_HCAP,), jnp.int32),
            pltpu.VMEM((_C, 2 * EMBED_DIM), jnp.float32),
            pltpu.VMEM((_C,), jnp.int32),
            pltpu.SemaphoreType.DMA,
            pltpu.SemaphoreType.DMA,
        ],
    )(body)


def kernel(latitudes, longitudes, lat_table, lon_table):
    k1 = _sc_call(_make_body(False))
    o1 = k1(latitudes, lat_table.T)
    k2 = _sc_call(_make_body(True))
    return k2(longitudes, lon_table.T, o1)


# speedup vs baseline: 2.2270x; 1.7130x over previous
import functools

import jax
import jax.numpy as jnp
from jax import lax
from jax.experimental import pallas as pl
from jax.experimental.pallas import tpu as pltpu
from jax.experimental.pallas import tpu_sc as plsc

LAT_BINS = 100000
LON_BINS = 100000
EMBED_DIM = 64
BATCH = 16384

_info = plsc.get_sparse_core_info()
_NC = _info.num_cores
_NS = _info.num_subcores
_NW = _NC * _NS
_BPW = BATCH // _NW
_CH = 128
_NCHUNK = _BPW // _CH


def _body(lat_idx_hbm, lon_idx_hbm, lat_t_hbm, lon_t_hbm, out_hbm,
          lat_idx_v, lon_idx_v, lat_v, lon_v, sem1, sem2):
    wid = lax.axis_index("s") * _NC + lax.axis_index("c")
    base = wid * _BPW
    pltpu.sync_copy(lat_idx_hbm.at[pl.ds(base, _BPW)], lat_idx_v)
    pltpu.sync_copy(lon_idx_hbm.at[pl.ds(base, _BPW)], lon_idx_v)
    for c in range(_NCHUNK):
        cp1 = pltpu.async_copy(
            lat_t_hbm.at[lat_idx_v.at[pl.ds(c * _CH, _CH)]], lat_v, sem1)
        cp2 = pltpu.async_copy(
            lon_t_hbm.at[lon_idx_v.at[pl.ds(c * _CH, _CH)]], lon_v, sem2)
        cp1.wait()
        pltpu.sync_copy(
            lat_v, out_hbm.at[pl.ds(base + c * _CH, _CH), pl.ds(0, EMBED_DIM)])
        cp2.wait()
        pltpu.sync_copy(
            lon_v,
            out_hbm.at[pl.ds(base + c * _CH, _CH), pl.ds(EMBED_DIM, EMBED_DIM)])


def kernel(latitudes, longitudes, lat_table, lon_table):
    mesh = plsc.VectorSubcoreMesh(core_axis_name="c", subcore_axis_name="s")
    k = functools.partial(
        pl.kernel,
        mesh=mesh,
        out_type=jax.ShapeDtypeStruct((BATCH, 2 * EMBED_DIM), jnp.float32),
        compiler_params=pltpu.CompilerParams(use_tc_tiling_on_sc=False),
        scratch_types=[
            pltpu.VMEM((_BPW,), jnp.int32),
            pltpu.VMEM((_BPW,), jnp.int32),
            pltpu.VMEM((_CH, EMBED_DIM), jnp.float32),
            pltpu.VMEM((_CH, EMBED_DIM), jnp.float32),
            pltpu.SemaphoreType.DMA,
            pltpu.SemaphoreType.DMA,
        ],
    )(_body)
    return k(latitudes, longitudes, lat_table, lon_table)
